# baseline (speedup 1.0000x reference)
import functools

import jax
import jax.numpy as jnp
from jax import lax
from jax.experimental import pallas as pl
from jax.experimental.pallas import tpu as pltpu
from jax.experimental.pallas import tpu_sc as plsc

K = 1000000
OUT_F = 16384
IN_F = 64
N = OUT_F * IN_F
CHUNK = 128
ROWS = N // CHUNK
NW = 32
ROWS_PW = ROWS // NW
GROUP = 8


def _sc_gather(mu_flat, rho_flat, idx2d):
    mesh = plsc.VectorSubcoreMesh(core_axis_name="c", subcore_axis_name="s")

    @functools.partial(
        pl.kernel,
        mesh=mesh,
        out_type=[
            jax.ShapeDtypeStruct((ROWS, CHUNK), jnp.float32),
            jax.ShapeDtypeStruct((ROWS, CHUNK), jnp.float32),
        ],
        scratch_types=[
            pltpu.VMEM((ROWS_PW, CHUNK), jnp.int32),
            pltpu.VMEM((ROWS_PW, CHUNK), jnp.float32),
            pltpu.VMEM((ROWS_PW, CHUNK), jnp.float32),
            pltpu.SemaphoreType.DMA,
        ],
    )
    def gather_kernel(mu_hbm, rho_hbm, idx_hbm, mu_out, rho_out,
                      idx_v, mu_v, rho_v, sem):
        wid = lax.axis_index("s") * 2 + lax.axis_index("c")
        row0 = wid * ROWS_PW
        pltpu.sync_copy(idx_hbm.at[pl.ds(row0, ROWS_PW)], idx_v)

        def body(g, carry):
            base = g * GROUP
            copies = []
            for b in range(GROUP):
                r = base + b
                copies.append(
                    pltpu.async_copy(mu_hbm.at[idx_v.at[r]], mu_v.at[r], sem))
                copies.append(
                    pltpu.async_copy(rho_hbm.at[idx_v.at[r]], rho_v.at[r], sem))
            for c in copies:
                c.wait()
            return carry

        lax.fori_loop(0, ROWS_PW // GROUP, body, 0)
        pltpu.sync_copy(mu_v, mu_out.at[pl.ds(row0, ROWS_PW)])
        pltpu.sync_copy(rho_v, rho_out.at[pl.ds(row0, ROWS_PW)])

    return gather_kernel(mu_flat, rho_flat, idx2d)


def _tc_combine(mu_g, rho_g, eps2d):
    BLK = 1024

    def body(mu_ref, rho_ref, eps_ref, out_ref):
        w = mu_ref[...] + jnp.log1p(jnp.exp(rho_ref[...])) * eps_ref[...]
        out_ref[...] = w.T

    return pl.pallas_call(
        body,
        grid=(OUT_F // BLK,),
        in_specs=[
            pl.BlockSpec((BLK, IN_F), lambda i: (i, 0)),
            pl.BlockSpec((BLK, IN_F), lambda i: (i, 0)),
            pl.BlockSpec((BLK, IN_F), lambda i: (i, 0)),
        ],
        out_specs=pl.BlockSpec((IN_F, BLK), lambda i: (0, i)),
        out_shape=jax.ShapeDtypeStruct((IN_F, OUT_F), jnp.float32),
    )(mu_g, rho_g, eps2d)


def kernel(weight_mu_share, weight_rho_share, eps_w, indices):
    mu_flat = weight_mu_share.reshape(K)
    rho_flat = weight_rho_share.reshape(K)
    idx2d = indices.reshape(ROWS, CHUNK)
    mu_g, rho_g = _sc_gather(mu_flat, rho_flat, idx2d)
    eps2d = eps_w.reshape(OUT_F, IN_F)
    return _tc_combine(mu_g.reshape(OUT_F, IN_F), rho_g.reshape(OUT_F, IN_F),
                       eps2d)


# speedup vs baseline: 1.0956x; 1.0956x over previous
import functools

import jax
import jax.numpy as jnp
from jax import lax
from jax.experimental import pallas as pl
from jax.experimental.pallas import tpu as pltpu
from jax.experimental.pallas import tpu_sc as plsc

K = 1000000
OUT_F = 16384
IN_F = 64
N = OUT_F * IN_F
CHUNK = 128
ROWS = N // CHUNK
NW = 32
ROWS_PW = ROWS // NW
GROUP = 8


N_PW = N // NW


def _sc_gather(mu_flat, rho_flat, idx_flat):
    mesh = plsc.VectorSubcoreMesh(core_axis_name="c", subcore_axis_name="s")

    @functools.partial(
        pl.kernel,
        mesh=mesh,
        out_type=[
            jax.ShapeDtypeStruct((N,), jnp.float32),
            jax.ShapeDtypeStruct((N,), jnp.float32),
        ],
        scratch_types=[
            pltpu.VMEM((N_PW,), jnp.int32),
            pltpu.VMEM((N_PW,), jnp.float32),
            pltpu.VMEM((N_PW,), jnp.float32),
            pltpu.SemaphoreType.DMA,
        ],
    )
    def gather_kernel(mu_hbm, rho_hbm, idx_hbm, mu_out, rho_out,
                      idx_v, mu_v, rho_v, sem):
        wid = lax.axis_index("s") * 2 + lax.axis_index("c")
        e0 = pl.multiple_of(wid * N_PW, N_PW)
        pltpu.sync_copy(idx_hbm.at[pl.ds(e0, N_PW)], idx_v)
        cm = pltpu.async_copy(mu_hbm.at[idx_v], mu_v, sem)
        cr = pltpu.async_copy(rho_hbm.at[idx_v], rho_v, sem)
        cm.wait()
        cr.wait()
        pltpu.sync_copy(mu_v, mu_out.at[pl.ds(e0, N_PW)])
        pltpu.sync_copy(rho_v, rho_out.at[pl.ds(e0, N_PW)])

    return gather_kernel(mu_flat, rho_flat, idx_flat)


def _tc_combine(mu_g, rho_g, eps2d):
    BLK = 1024

    def body(mu_ref, rho_ref, eps_ref, out_ref):
        w = mu_ref[...] + jnp.log1p(jnp.exp(rho_ref[...])) * eps_ref[...]
        out_ref[...] = w.T

    return pl.pallas_call(
        body,
        grid=(OUT_F // BLK,),
        in_specs=[
            pl.BlockSpec((BLK, IN_F), lambda i: (i, 0)),
            pl.BlockSpec((BLK, IN_F), lambda i: (i, 0)),
            pl.BlockSpec((BLK, IN_F), lambda i: (i, 0)),
        ],
        out_specs=pl.BlockSpec((IN_F, BLK), lambda i: (0, i)),
        out_shape=jax.ShapeDtypeStruct((IN_F, OUT_F), jnp.float32),
    )(mu_g, rho_g, eps2d)


def kernel(weight_mu_share, weight_rho_share, eps_w, indices):
    mu_flat = weight_mu_share.reshape(K)
    rho_flat = weight_rho_share.reshape(K)
    idx_flat = indices.reshape(N)
    mu_g, rho_g = _sc_gather(mu_flat, rho_flat, idx_flat)
    eps2d = eps_w.reshape(OUT_F, IN_F)
    return _tc_combine(mu_g.reshape(OUT_F, IN_F), rho_g.reshape(OUT_F, IN_F),
                       eps2d)


# speedup vs baseline: 2.1042x; 1.9205x over previous
import functools

import jax
import jax.numpy as jnp
from jax import lax
from jax.experimental import pallas as pl
from jax.experimental.pallas import tpu as pltpu
from jax.experimental.pallas import tpu_sc as plsc

K = 1000000
OUT_F = 16384
IN_F = 64
N = OUT_F * IN_F
NW = 32
N_PW = N // NW


def _sc_gather(mu2d, rho2d, idx_flat):
    mesh = plsc.VectorSubcoreMesh(core_axis_name="c", subcore_axis_name="s")

    @functools.partial(
        pl.kernel,
        mesh=mesh,
        out_type=[
            jax.ShapeDtypeStruct((N,), jnp.float32),
            jax.ShapeDtypeStruct((N,), jnp.float32),
        ],
        scratch_types=[
            pltpu.VMEM((N_PW,), jnp.int32),
            pltpu.VMEM((N_PW,), jnp.float32),
            pltpu.VMEM((N_PW,), jnp.float32),
            pltpu.SemaphoreType.DMA,
        ],
    )
    def gather_kernel(mu_hbm, rho_hbm, idx_hbm, mu_out, rho_out,
                      idx_v, mu_v, rho_v, sem):
        wid = lax.axis_index("s") * 2 + lax.axis_index("c")
        e0 = pl.multiple_of(wid * N_PW, N_PW)
        pltpu.sync_copy(idx_hbm.at[pl.ds(e0, N_PW)], idx_v)
        cm = pltpu.async_copy(mu_hbm.at[0].at[idx_v], mu_v, sem)
        cr = pltpu.async_copy(rho_hbm.at[0].at[idx_v], rho_v, sem)
        cm.wait()
        cr.wait()
        pltpu.sync_copy(mu_v, mu_out.at[pl.ds(e0, N_PW)])
        pltpu.sync_copy(rho_v, rho_out.at[pl.ds(e0, N_PW)])

    return gather_kernel(mu2d, rho2d, idx_flat)


def _tc_combine(mu_t, rho_t, eps_t):
    BLK = 8

    def body(mu_ref, rho_ref, eps_ref, out_ref):
        out_ref[...] = (mu_ref[...]
                        + jnp.log1p(jnp.exp(rho_ref[...])) * eps_ref[...])

    return pl.pallas_call(
        body,
        grid=(IN_F // BLK,),
        in_specs=[
            pl.BlockSpec((BLK, OUT_F), lambda i: (i, 0)),
            pl.BlockSpec((BLK, OUT_F), lambda i: (i, 0)),
            pl.BlockSpec((BLK, OUT_F), lambda i: (i, 0)),
        ],
        out_specs=pl.BlockSpec((BLK, OUT_F), lambda i: (i, 0)),
        out_shape=jax.ShapeDtypeStruct((IN_F, OUT_F), jnp.float32),
    )(mu_t, rho_t, eps_t)


def kernel(weight_mu_share, weight_rho_share, eps_w, indices):
    idx_t = jnp.transpose(indices[0], (1, 0)).reshape(N)
    eps_t = jnp.transpose(eps_w[0], (1, 0))
    mu_g, rho_g = _sc_gather(weight_mu_share, weight_rho_share, idx_t)
    return _tc_combine(mu_g.reshape(IN_F, OUT_F), rho_g.reshape(IN_F, OUT_F),
                       eps_t)


# speedup vs baseline: 2.2635x; 1.0757x over previous
import functools

import jax
import jax.numpy as jnp
from jax import lax
from jax.experimental import pallas as pl
from jax.experimental.pallas import tpu as pltpu
from jax.experimental.pallas import tpu_sc as plsc

K = 1000000
OUT_F = 16384
IN_F = 64
N = OUT_F * IN_F
NW = 32
N_PW = N // NW


def _sc_gather(mu2d, rho2d, idx_flat):
    mesh = plsc.VectorSubcoreMesh(core_axis_name="c", subcore_axis_name="s")

    @functools.partial(
        pl.kernel,
        mesh=mesh,
        out_type=[
            jax.ShapeDtypeStruct((IN_F, OUT_F), jnp.float32),
            jax.ShapeDtypeStruct((IN_F, OUT_F), jnp.float32),
        ],
        scratch_types=[
            pltpu.VMEM((N_PW,), jnp.int32),
            pltpu.VMEM((N_PW,), jnp.float32),
            pltpu.VMEM((N_PW,), jnp.float32),
            pltpu.SemaphoreType.DMA,
        ],
    )
    def gather_kernel(mu_hbm, rho_hbm, idx_hbm, mu_out, rho_out,
                      idx_v, mu_v, rho_v, sem):
        wid = lax.axis_index("s") * 2 + lax.axis_index("c")
        e0 = pl.multiple_of(wid * N_PW, N_PW)
        pltpu.sync_copy(idx_hbm.at[pl.ds(e0, N_PW)], idx_v)
        cm = pltpu.async_copy(mu_hbm.at[0].at[idx_v], mu_v, sem)
        cr = pltpu.async_copy(rho_hbm.at[0].at[idx_v], rho_v, sem)
        cm.wait()
        cr.wait()
        row = 2 * wid
        pltpu.sync_copy(mu_v.at[pl.ds(0, OUT_F)], mu_out.at[row])
        pltpu.sync_copy(mu_v.at[pl.ds(OUT_F, OUT_F)], mu_out.at[row + 1])
        pltpu.sync_copy(rho_v.at[pl.ds(0, OUT_F)], rho_out.at[row])
        pltpu.sync_copy(rho_v.at[pl.ds(OUT_F, OUT_F)], rho_out.at[row + 1])

    return gather_kernel(mu2d, rho2d, idx_flat)


def _tc_combine(mu_t, rho_t, eps_t):
    BLK = 8

    def body(mu_ref, rho_ref, eps_ref, out_ref):
        out_ref[...] = (mu_ref[...]
                        + jnp.log1p(jnp.exp(rho_ref[...])) * eps_ref[...])

    return pl.pallas_call(
        body,
        grid=(IN_F // BLK,),
        in_specs=[
            pl.BlockSpec((BLK, OUT_F), lambda i: (i, 0)),
            pl.BlockSpec((BLK, OUT_F), lambda i: (i, 0)),
            pl.BlockSpec((BLK, OUT_F), lambda i: (i, 0)),
        ],
        out_specs=pl.BlockSpec((BLK, OUT_F), lambda i: (i, 0)),
        out_shape=jax.ShapeDtypeStruct((IN_F, OUT_F), jnp.float32),
    )(mu_t, rho_t, eps_t)


def kernel(weight_mu_share, weight_rho_share, eps_w, indices):
    idx_t = jnp.transpose(indices[0], (1, 0)).reshape(N)
    eps_t = jnp.transpose(eps_w[0], (1, 0))
    mu_g, rho_g = _sc_gather(weight_mu_share, weight_rho_share, idx_t)
    return _tc_combine(mu_g, rho_g, eps_t)


# speedup vs baseline: 2.6605x; 1.1754x over previous
import functools

import jax
import jax.numpy as jnp
from jax import lax
from jax.experimental import pallas as pl
from jax.experimental.pallas import tpu as pltpu
from jax.experimental.pallas import tpu_sc as plsc

K = 1000000
OUT_F = 16384
IN_F = 64
N = OUT_F * IN_F
NW = 32
N_PW = N // NW
CH_A = 31256
CH_LAST = K - 31 * CH_A
CH_PAD = 31264
MASK_HI = -65536


def _sc_pack(mu2d, rho2d):
    mesh = plsc.VectorSubcoreMesh(core_axis_name="c", subcore_axis_name="s")

    @functools.partial(
        pl.kernel,
        mesh=mesh,
        out_type=jax.ShapeDtypeStruct((K,), jnp.int32),
        scratch_types=[
            pltpu.VMEM((CH_PAD,), jnp.int32),
            pltpu.VMEM((CH_PAD,), jnp.int32),
            pltpu.VMEM((CH_PAD,), jnp.int32),
        ],
    )
    def pack_kernel(mu_hbm, rho_hbm, packed_out, mu_v, rho_v, out_v):
        wid = lax.axis_index("s") * 2 + lax.axis_index("c")
        base = wid * CH_A

        @pl.when(wid < NW - 1)
        def _():
            pltpu.sync_copy(mu_hbm.at[0].at[pl.ds(base, CH_A)],
                            mu_v.at[pl.ds(0, CH_A)])
            pltpu.sync_copy(rho_hbm.at[0].at[pl.ds(base, CH_A)],
                            rho_v.at[pl.ds(0, CH_A)])

        @pl.when(wid == NW - 1)
        def _():
            pltpu.sync_copy(mu_hbm.at[0].at[pl.ds(base, CH_LAST)],
                            mu_v.at[pl.ds(0, CH_LAST)])
            pltpu.sync_copy(rho_hbm.at[0].at[pl.ds(base, CH_LAST)],
                            rho_v.at[pl.ds(0, CH_LAST)])

        def body(j, carry):
            off = j * 16
            m = mu_v[pl.ds(off, 16)] + 0x8000
            r = rho_v[pl.ds(off, 16)] + 0x8000
            packed = (m & MASK_HI) | lax.shift_right_logical(r, 16)
            out_v[pl.ds(off, 16)] = packed
            return carry

        lax.fori_loop(0, CH_PAD // 16, body, 0)

        @pl.when(wid < NW - 1)
        def _():
            pltpu.sync_copy(out_v.at[pl.ds(0, CH_A)],
                            packed_out.at[pl.ds(base, CH_A)])

        @pl.when(wid == NW - 1)
        def _():
            pltpu.sync_copy(out_v.at[pl.ds(0, CH_LAST)],
                            packed_out.at[pl.ds(base, CH_LAST)])

    return pack_kernel(lax.bitcast_convert_type(mu2d, jnp.int32),
                       lax.bitcast_convert_type(rho2d, jnp.int32))


def _sc_gather(packed_tab, idx_flat):
    mesh = plsc.VectorSubcoreMesh(core_axis_name="c", subcore_axis_name="s")

    @functools.partial(
        pl.kernel,
        mesh=mesh,
        out_type=jax.ShapeDtypeStruct((IN_F, OUT_F), jnp.int32),
        scratch_types=[
            pltpu.VMEM((N_PW,), jnp.int32),
            pltpu.VMEM((N_PW,), jnp.int32),
            pltpu.SemaphoreType.DMA,
        ],
    )
    def gather_kernel(tab_hbm, idx_hbm, packed_out, idx_v, g_v, sem):
        wid = lax.axis_index("s") * 2 + lax.axis_index("c")
        e0 = pl.multiple_of(wid * N_PW, N_PW)
        pltpu.sync_copy(idx_hbm.at[pl.ds(e0, N_PW)], idx_v)
        pltpu.async_copy(tab_hbm.at[idx_v], g_v, sem).wait()
        row = 2 * wid
        pltpu.sync_copy(g_v.at[pl.ds(0, OUT_F)], packed_out.at[row])
        pltpu.sync_copy(g_v.at[pl.ds(OUT_F, OUT_F)], packed_out.at[row + 1])

    return gather_kernel(packed_tab, idx_flat)


def _tc_combine(packed_g, eps_t):
    BLK = 8

    def body(p_ref, eps_ref, out_ref):
        p = p_ref[...]
        mu = lax.bitcast_convert_type(p & MASK_HI, jnp.float32)
        rho = lax.bitcast_convert_type(lax.shift_left(p, 16), jnp.float32)
        out_ref[...] = mu + jnp.log1p(jnp.exp(rho)) * eps_ref[...]

    return pl.pallas_call(
        body,
        grid=(IN_F // BLK,),
        in_specs=[
            pl.BlockSpec((BLK, OUT_F), lambda i: (i, 0)),
            pl.BlockSpec((BLK, OUT_F), lambda i: (i, 0)),
        ],
        out_specs=pl.BlockSpec((BLK, OUT_F), lambda i: (i, 0)),
        out_shape=jax.ShapeDtypeStruct((IN_F, OUT_F), jnp.float32),
    )(packed_g, eps_t)


def kernel(weight_mu_share, weight_rho_share, eps_w, indices):
    idx_t = jnp.transpose(indices[0], (1, 0)).reshape(N)
    eps_t = jnp.transpose(eps_w[0], (1, 0))
    packed_tab = _sc_pack(weight_mu_share, weight_rho_share)
    packed_g = _sc_gather(packed_tab, idx_t)
    return _tc_combine(packed_g, eps_t)


# speedup vs baseline: 3.1591x; 1.1874x over previous
import functools

import jax
import jax.numpy as jnp
from jax import lax
from jax.experimental import pallas as pl
from jax.experimental.pallas import tpu as pltpu
from jax.experimental.pallas import tpu_sc as plsc

K = 1000000
OUT_F = 16384
IN_F = 64
N = OUT_F * IN_F
NW = 32
N_PW = N // NW
CH_A = 31256
CH_LAST = K - 31 * CH_A
CH_PAD = 31296
MASK_HI = -65536


def _sc_pack(mu2d, rho2d):
    mesh = plsc.VectorSubcoreMesh(core_axis_name="c", subcore_axis_name="s")

    @functools.partial(
        pl.kernel,
        mesh=mesh,
        out_type=jax.ShapeDtypeStruct((K,), jnp.int32),
        scratch_types=[
            pltpu.VMEM((CH_PAD,), jnp.float32),
            pltpu.VMEM((CH_PAD,), jnp.float32),
            pltpu.VMEM((CH_PAD,), jnp.int32),
        ],
    )
    def pack_kernel(mu_hbm, rho_hbm, packed_out, mu_v, rho_v, out_v):
        wid = lax.axis_index("s") * 2 + lax.axis_index("c")
        base = wid * CH_A

        @pl.when(wid < NW - 1)
        def _():
            pltpu.sync_copy(mu_hbm.at[0].at[pl.ds(base, CH_A)],
                            mu_v.at[pl.ds(0, CH_A)])
            pltpu.sync_copy(rho_hbm.at[0].at[pl.ds(base, CH_A)],
                            rho_v.at[pl.ds(0, CH_A)])

        @pl.when(wid == NW - 1)
        def _():
            pltpu.sync_copy(mu_hbm.at[0].at[pl.ds(base, CH_LAST)],
                            mu_v.at[pl.ds(0, CH_LAST)])
            pltpu.sync_copy(rho_hbm.at[0].at[pl.ds(base, CH_LAST)],
                            rho_v.at[pl.ds(0, CH_LAST)])

        def body(j, carry):
            for u in range(4):
                off = j * 64 + u * 16
                m = lax.bitcast_convert_type(mu_v[pl.ds(off, 16)], jnp.int32) + 0x8000
                r = lax.bitcast_convert_type(rho_v[pl.ds(off, 16)], jnp.int32) + 0x8000
                packed = (m & MASK_HI) | lax.shift_right_logical(r, 16)
                out_v[pl.ds(off, 16)] = packed
            return carry

        lax.fori_loop(0, CH_PAD // 64, body, 0)

        @pl.when(wid < NW - 1)
        def _():
            pltpu.sync_copy(out_v.at[pl.ds(0, CH_A)],
                            packed_out.at[pl.ds(base, CH_A)])

        @pl.when(wid == NW - 1)
        def _():
            pltpu.sync_copy(out_v.at[pl.ds(0, CH_LAST)],
                            packed_out.at[pl.ds(base, CH_LAST)])

    return pack_kernel(mu2d, rho2d)


def _sc_gather(packed_tab, idx_flat):
    mesh = plsc.VectorSubcoreMesh(core_axis_name="c", subcore_axis_name="s")

    @functools.partial(
        pl.kernel,
        mesh=mesh,
        out_type=jax.ShapeDtypeStruct((IN_F, OUT_F), jnp.int32),
        scratch_types=[
            pltpu.VMEM((N_PW,), jnp.int32),
            pltpu.VMEM((N_PW,), jnp.int32),
            pltpu.SemaphoreType.DMA,
        ],
    )
    def gather_kernel(tab_hbm, idx_hbm, packed_out, idx_v, g_v, sem):
        wid = lax.axis_index("s") * 2 + lax.axis_index("c")
        e0 = pl.multiple_of(wid * N_PW, N_PW)
        pltpu.sync_copy(idx_hbm.at[pl.ds(e0, N_PW)], idx_v)
        pltpu.async_copy(tab_hbm.at[idx_v], g_v, sem).wait()
        row = 2 * wid
        pltpu.sync_copy(g_v.at[pl.ds(0, OUT_F)], packed_out.at[row])
        pltpu.sync_copy(g_v.at[pl.ds(OUT_F, OUT_F)], packed_out.at[row + 1])

    return gather_kernel(packed_tab, idx_flat)


def _tc_combine(packed_g, eps_t):
    BLK = 16

    def body(p_ref, eps_ref, out_ref):
        p = p_ref[...]
        mu = lax.bitcast_convert_type(p & MASK_HI, jnp.float32)
        rho = lax.bitcast_convert_type(lax.shift_left(p, 16), jnp.float32)
        out_ref[...] = mu + jnp.log1p(jnp.exp(rho)) * eps_ref[...]

    return pl.pallas_call(
        body,
        grid=(IN_F // BLK,),
        in_specs=[
            pl.BlockSpec((BLK, OUT_F), lambda i: (i, 0)),
            pl.BlockSpec((BLK, OUT_F), lambda i: (i, 0)),
        ],
        out_specs=pl.BlockSpec((BLK, OUT_F), lambda i: (i, 0)),
        out_shape=jax.ShapeDtypeStruct((IN_F, OUT_F), jnp.float32),
    )(packed_g, eps_t)


def kernel(weight_mu_share, weight_rho_share, eps_w, indices):
    idx_t = jnp.transpose(indices[0], (1, 0)).reshape(N)
    eps_t = jnp.transpose(eps_w[0], (1, 0))
    packed_tab = _sc_pack(weight_mu_share, weight_rho_share)
    packed_g = _sc_gather(packed_tab, idx_t)
    return _tc_combine(packed_g, eps_t)
